# baseline (speedup 1.0000x reference)
import functools

import jax
import jax.numpy as jnp
from jax.experimental import pallas as pl
from jax.experimental.pallas import tpu as pltpu

EMB_DIM = 128
_VMEM_PARAMS = pltpu.CompilerParams(vmem_limit_bytes=110 * 1024 * 1024)


def _intra_block(adj, mat, emb):
    r = jnp.sum(emb, axis=1)
    logits = mat * r[None, :]
    m = jnp.max(logits, axis=1, keepdims=True)
    e = jnp.exp(logits - m)
    s = e / jnp.sum(e, axis=1, keepdims=True)
    w = s * adj
    w = w / (jnp.sum(w, axis=1, keepdims=True) + 1e-8)
    return jnp.dot(w, emb, preferred_element_type=jnp.float32)


def _gate(e, W, b):
    return jnp.exp(jnp.sum(e * W, axis=1, keepdims=True) + b)


def _inter(W, b, e0, e1, e2, e3):
    g0 = _gate(e0, W, b)
    g1 = _gate(e1, W, b)
    g2 = _gate(e2, W, b)
    g3 = _gate(e3, W, b)
    s = g0 + g1 + g2 + g3
    return (g0 / s) * e0 + (g1 / s) * e1 + (g2 / s) * e2 + (g3 / s) * e3


def _item_gates(i, br, item_ref, avp_ref, avcb_ref, avcm_ref, pri_ref,
                cb_ref, cm_ref, mvp_ref, mvcb_ref, mvcm_ref, W_ref, b_ref):
    e0 = item_ref[pl.ds(i * br, br), :]
    e1 = _intra_block(avp_ref[...], mvp_ref[...], pri_ref[...])
    e2 = _intra_block(avcb_ref[...], mvcb_ref[...], cb_ref[...])
    e3 = _intra_block(avcm_ref[...], mvcm_ref[...], cm_ref[...])
    return _inter(W_ref[...], b_ref[...], e0, e1, e2, e3)


def _item_kernel_l1(adj_ref, avp_ref, avcb_ref, avcm_ref, item_ref, pri_ref,
                    cb_ref, cm_ref, mvp_ref, mvcb_ref, mvcm_ref, W_ref,
                    b_ref, out_ref, q_ref, s_ref, *, br):
    i = pl.program_id(0)
    gated = _item_gates(i, br, item_ref, avp_ref, avcb_ref, avcm_ref,
                        pri_ref, cb_ref, cm_ref, mvp_ref, mvcb_ref, mvcm_ref,
                        W_ref, b_ref)
    a = adj_ref[...]
    m = jnp.maximum(jnp.max(a, axis=1, keepdims=True), 1e-30)
    inv = 127.0 / m
    q_ref[...] = jnp.minimum(jnp.round(a * inv), 127.0).astype(jnp.int8)
    s_ref[...] = m * (1.0 / 127.0)
    big = jnp.dot(a.astype(jnp.bfloat16), item_ref[...].astype(jnp.bfloat16),
                  preferred_element_type=jnp.float32)
    out_ref[...] = gated + big


def _item_update_l1(adjacency, a_vp, a_vcb, a_vcm, item, pri, cb, cm,
                    m_vp, m_vcb, m_vcm, W, b):
    n = item.shape[0]
    br = 400 if n % 400 == 0 else n
    grid = (n // br,)
    return pl.pallas_call(
        functools.partial(_item_kernel_l1, br=br),
        grid=grid,
        in_specs=[
            pl.BlockSpec((br, n), lambda i: (i, 0)),
            pl.BlockSpec((br, a_vp.shape[1]), lambda i: (i, 0)),
            pl.BlockSpec((br, a_vcb.shape[1]), lambda i: (i, 0)),
            pl.BlockSpec((br, a_vcm.shape[1]), lambda i: (i, 0)),
            pl.BlockSpec((n, EMB_DIM), lambda i: (0, 0)),
            pl.BlockSpec(pri.shape, lambda i: (0, 0)),
            pl.BlockSpec(cb.shape, lambda i: (0, 0)),
            pl.BlockSpec(cm.shape, lambda i: (0, 0)),
            pl.BlockSpec((br, 1), lambda i: (i, 0)),
            pl.BlockSpec((br, 1), lambda i: (i, 0)),
            pl.BlockSpec((br, 1), lambda i: (i, 0)),
            pl.BlockSpec((1, EMB_DIM), lambda i: (0, 0)),
            pl.BlockSpec((1, 1), lambda i: (0, 0)),
        ],
        out_specs=(
            pl.BlockSpec((br, EMB_DIM), lambda i: (i, 0)),
            pl.BlockSpec((br, n), lambda i: (i, 0)),
            pl.BlockSpec((br, 1), lambda i: (i, 0)),
        ),
        out_shape=(
            jax.ShapeDtypeStruct((n, EMB_DIM), jnp.float32),
            jax.ShapeDtypeStruct((n, n), jnp.int8),
            jax.ShapeDtypeStruct((n, 1), jnp.float32),
        ),
        compiler_params=_VMEM_PARAMS,
    )(adjacency, a_vp, a_vcb, a_vcm, item, pri, cb, cm, m_vp, m_vcb, m_vcm,
      W, b)


def _item_kernel_f32(adj_ref, avp_ref, avcb_ref, avcm_ref, item_ref, pri_ref,
                     cb_ref, cm_ref, mvp_ref, mvcb_ref, mvcm_ref, W_ref,
                     b_ref, out_ref, *, br):
    i = pl.program_id(0)
    gated = _item_gates(i, br, item_ref, avp_ref, avcb_ref, avcm_ref,
                        pri_ref, cb_ref, cm_ref, mvp_ref, mvcb_ref, mvcm_ref,
                        W_ref, b_ref)
    big = jnp.dot(adj_ref[...].astype(jnp.bfloat16),
                  item_ref[...].astype(jnp.bfloat16),
                  preferred_element_type=jnp.float32)
    out_ref[...] = gated + big


def _item_update_f32(adjacency, a_vp, a_vcb, a_vcm, item, pri, cb, cm,
                     m_vp, m_vcb, m_vcm, W, b):
    n = item.shape[0]
    br = 400 if n % 400 == 0 else n
    grid = (n // br,)
    return pl.pallas_call(
        functools.partial(_item_kernel_f32, br=br),
        grid=grid,
        in_specs=[
            pl.BlockSpec((br, n), lambda i: (i, 0)),
            pl.BlockSpec((br, a_vp.shape[1]), lambda i: (i, 0)),
            pl.BlockSpec((br, a_vcb.shape[1]), lambda i: (i, 0)),
            pl.BlockSpec((br, a_vcm.shape[1]), lambda i: (i, 0)),
            pl.BlockSpec((n, EMB_DIM), lambda i: (0, 0)),
            pl.BlockSpec(pri.shape, lambda i: (0, 0)),
            pl.BlockSpec(cb.shape, lambda i: (0, 0)),
            pl.BlockSpec(cm.shape, lambda i: (0, 0)),
            pl.BlockSpec((br, 1), lambda i: (i, 0)),
            pl.BlockSpec((br, 1), lambda i: (i, 0)),
            pl.BlockSpec((br, 1), lambda i: (i, 0)),
            pl.BlockSpec((1, EMB_DIM), lambda i: (0, 0)),
            pl.BlockSpec((1, 1), lambda i: (0, 0)),
        ],
        out_specs=pl.BlockSpec((br, EMB_DIM), lambda i: (i, 0)),
        out_shape=jax.ShapeDtypeStruct((n, EMB_DIM), jnp.float32),
        compiler_params=_VMEM_PARAMS,
    )(adjacency, a_vp, a_vcb, a_vcm, item, pri, cb, cm, m_vp, m_vcb, m_vcm,
      W, b)


def _item_kernel_l2(q_ref, s_ref, itq_ref, t_ref, avp_ref, avcb_ref,
                    avcm_ref, item_ref, pri_ref, cb_ref, cm_ref, mvp_ref,
                    mvcb_ref, mvcm_ref, W_ref, b_ref, out_ref, *, br):
    i = pl.program_id(0)
    gated = _item_gates(i, br, item_ref, avp_ref, avcb_ref, avcm_ref,
                        pri_ref, cb_ref, cm_ref, mvp_ref, mvcb_ref, mvcm_ref,
                        W_ref, b_ref)
    acc = jnp.dot(q_ref[...], itq_ref[...],
                  preferred_element_type=jnp.int32)
    big = acc.astype(jnp.float32) * s_ref[...] * t_ref[...]
    out_ref[...] = gated + big


def _item_update_l2(q_adj, s_adj, itq, t, a_vp, a_vcb, a_vcm, item, pri, cb,
                    cm, m_vp, m_vcb, m_vcm, W, b):
    n = item.shape[0]
    br = 400 if n % 400 == 0 else n
    grid = (n // br,)
    return pl.pallas_call(
        functools.partial(_item_kernel_l2, br=br),
        grid=grid,
        in_specs=[
            pl.BlockSpec((br, n), lambda i: (i, 0)),
            pl.BlockSpec((br, 1), lambda i: (i, 0)),
            pl.BlockSpec((n, EMB_DIM), lambda i: (0, 0)),
            pl.BlockSpec((1, EMB_DIM), lambda i: (0, 0)),
            pl.BlockSpec((br, a_vp.shape[1]), lambda i: (i, 0)),
            pl.BlockSpec((br, a_vcb.shape[1]), lambda i: (i, 0)),
            pl.BlockSpec((br, a_vcm.shape[1]), lambda i: (i, 0)),
            pl.BlockSpec((n, EMB_DIM), lambda i: (0, 0)),
            pl.BlockSpec(pri.shape, lambda i: (0, 0)),
            pl.BlockSpec(cb.shape, lambda i: (0, 0)),
            pl.BlockSpec(cm.shape, lambda i: (0, 0)),
            pl.BlockSpec((br, 1), lambda i: (i, 0)),
            pl.BlockSpec((br, 1), lambda i: (i, 0)),
            pl.BlockSpec((br, 1), lambda i: (i, 0)),
            pl.BlockSpec((1, EMB_DIM), lambda i: (0, 0)),
            pl.BlockSpec((1, 1), lambda i: (0, 0)),
        ],
        out_specs=pl.BlockSpec((br, EMB_DIM), lambda i: (i, 0)),
        out_shape=jax.ShapeDtypeStruct((n, EMB_DIM), jnp.float32),
        compiler_params=_VMEM_PARAMS,
    )(q_adj, s_adj, itq, t, a_vp, a_vcb, a_vcm, item, pri, cb, cm,
      m_vp, m_vcb, m_vcm, W, b)


def _acc_intra(adj_blk, mat, r, emb_blk, n_acc, s_acc, d_acc):
    e = jnp.exp(mat * r[None, :])
    w = e * adj_blk
    n_acc[...] += jnp.dot(w, emb_blk, preferred_element_type=jnp.float32)
    s_acc[...] += jnp.sum(w, axis=1, keepdims=True)
    d_acc[...] += jnp.sum(e, axis=1, keepdims=True)


def _finish_intra(n_acc, s_acc, d_acc, n_pad):
    return n_acc[...] / (s_acc[...] + 1e-8 * (d_acc[...] - n_pad))


def _pcc_kernel(apv_ref, acbv_ref, acmv_ref, item_ref,
                apcb_ref, apcm_ref, acbp_ref, acbcm_ref, acmp_ref, acmcb_ref,
                pri_ref, cb_ref, cm_ref,
                mpv_ref, mcbv_ref, mcmv_ref, mpcb_ref, mpcm_ref, mcbp_ref,
                mcbcm_ref, mcmp_ref, mcmcb_ref,
                Wp_ref, bp_ref, Wcb_ref, bcb_ref, Wcm_ref, bcm_ref,
                pr_out, cbn_out, cmn_out,
                np_a, sp_a, dp_a, ncb_a, scb_a, dcb_a, ncm_a, scm_a, dcm_a,
                *, nk, n_pad):
    k = pl.program_id(0)

    @pl.when(k == 0)
    def _init():
        for a in (np_a, sp_a, dp_a, ncb_a, scb_a, dcb_a, ncm_a, scm_a, dcm_a):
            a[...] = jnp.zeros_like(a)

    itb = item_ref[...]
    r = jnp.sum(itb, axis=1)
    _acc_intra(apv_ref[...], mpv_ref[...], r, itb, np_a, sp_a, dp_a)
    _acc_intra(acbv_ref[...], mcbv_ref[...], r, itb, ncb_a, scb_a, dcb_a)
    _acc_intra(acmv_ref[...], mcmv_ref[...], r, itb, ncm_a, scm_a, dcm_a)

    @pl.when(k == nk - 1)
    def _final():
        p = pri_ref[...]
        c_b = cb_ref[...]
        c_m = cm_ref[...]
        pr_out[...] = _inter(
            Wp_ref[...], bp_ref[...], p,
            _finish_intra(np_a, sp_a, dp_a, n_pad),
            _intra_block(apcb_ref[...], mpcb_ref[...], c_b),
            _intra_block(apcm_ref[...], mpcm_ref[...], c_m))
        cbn_out[...] = _inter(
            Wcb_ref[...], bcb_ref[...], c_b,
            _intra_block(acbp_ref[...], mcbp_ref[...], p),
            _finish_intra(ncb_a, scb_a, dcb_a, n_pad),
            _intra_block(acbcm_ref[...], mcbcm_ref[...], c_m))
        cmn_out[...] = _inter(
            Wcm_ref[...], bcm_ref[...], c_m,
            _intra_block(acmp_ref[...], mcmp_ref[...], p),
            _finish_intra(ncm_a, scm_a, dcm_a, n_pad),
            _intra_block(acmcb_ref[...], mcmcb_ref[...], c_b))


def _pcc_update(a_pv, a_pcb, a_pcm, a_cbp, a_cbv, a_cbcm, a_cmp, a_cmv,
                a_cmcb, item, pri, cb, cm, m_pv, m_pcb, m_pcm, m_cbp, m_cbv,
                m_cbcm, m_cmp, m_cmv, m_cmcb, Wp, bp, Wcb, bcb, Wcm, bcm,
                n_pad):
    n = item.shape[0]
    bk = 2048
    nk = n // bk
    np_, ncb, ncm = pri.shape[0], cb.shape[0], cm.shape[0]
    full = lambda x: pl.BlockSpec(x.shape, lambda k: (0, 0))
    return pl.pallas_call(
        functools.partial(_pcc_kernel, nk=nk, n_pad=n_pad),
        grid=(nk,),
        in_specs=[
            pl.BlockSpec((np_, bk), lambda k: (0, k)),
            pl.BlockSpec((ncb, bk), lambda k: (0, k)),
            pl.BlockSpec((ncm, bk), lambda k: (0, k)),
            pl.BlockSpec((bk, EMB_DIM), lambda k: (k, 0)),
            full(a_pcb), full(a_pcm), full(a_cbp), full(a_cbcm),
            full(a_cmp), full(a_cmcb), full(pri), full(cb), full(cm),
            full(m_pv), full(m_cbv), full(m_cmv), full(m_pcb), full(m_pcm),
            full(m_cbp), full(m_cbcm), full(m_cmp), full(m_cmcb),
            full(Wp), full(bp), full(Wcb), full(bcb), full(Wcm), full(bcm),
        ],
        out_specs=(
            pl.BlockSpec((np_, EMB_DIM), lambda k: (0, 0)),
            pl.BlockSpec((ncb, EMB_DIM), lambda k: (0, 0)),
            pl.BlockSpec((ncm, EMB_DIM), lambda k: (0, 0)),
        ),
        out_shape=(
            jax.ShapeDtypeStruct((np_, EMB_DIM), jnp.float32),
            jax.ShapeDtypeStruct((ncb, EMB_DIM), jnp.float32),
            jax.ShapeDtypeStruct((ncm, EMB_DIM), jnp.float32),
        ),
        scratch_shapes=[
            pltpu.VMEM((np_, EMB_DIM), jnp.float32),
            pltpu.VMEM((np_, 1), jnp.float32),
            pltpu.VMEM((np_, 1), jnp.float32),
            pltpu.VMEM((ncb, EMB_DIM), jnp.float32),
            pltpu.VMEM((ncb, 1), jnp.float32),
            pltpu.VMEM((ncb, 1), jnp.float32),
            pltpu.VMEM((ncm, EMB_DIM), jnp.float32),
            pltpu.VMEM((ncm, 1), jnp.float32),
            pltpu.VMEM((ncm, 1), jnp.float32),
        ],
        compiler_params=_VMEM_PARAMS,
    )(a_pv, a_cbv, a_cmv, item, a_pcb, a_pcm, a_cbp, a_cbcm, a_cmp, a_cmcb,
      pri, cb, cm, m_pv, m_cbv, m_cmv, m_pcb, m_pcm, m_cbp, m_cbcm,
      m_cmp, m_cmcb, Wp, bp, Wcb, bcb, Wcm, bcm)


def _pr_kernel(apv_ref, item_ref, apcb_ref, apcm_ref, pri_ref, cb_ref,
               cm_ref, mpv_ref, mpcb_ref, mpcm_ref, Wp_ref, bp_ref, pr_out,
               np_a, sp_a, dp_a, *, nk, n_pad):
    k = pl.program_id(0)

    @pl.when(k == 0)
    def _init():
        for a in (np_a, sp_a, dp_a):
            a[...] = jnp.zeros_like(a)

    itb = item_ref[...]
    r = jnp.sum(itb, axis=1)
    _acc_intra(apv_ref[...], mpv_ref[...], r, itb, np_a, sp_a, dp_a)

    @pl.when(k == nk - 1)
    def _final():
        pr_out[...] = _inter(
            Wp_ref[...], bp_ref[...], pri_ref[...],
            _finish_intra(np_a, sp_a, dp_a, n_pad),
            _intra_block(apcb_ref[...], mpcb_ref[...], cb_ref[...]),
            _intra_block(apcm_ref[...], mpcm_ref[...], cm_ref[...]))


def _pr_update(a_pv, a_pcb, a_pcm, item, pri, cb, cm, m_pv, m_pcb, m_pcm,
               Wp, bp, n_pad):
    n = item.shape[0]
    bk = 2048
    nk = n // bk
    np_ = pri.shape[0]
    full = lambda x: pl.BlockSpec(x.shape, lambda k: (0, 0))
    return pl.pallas_call(
        functools.partial(_pr_kernel, nk=nk, n_pad=n_pad),
        grid=(nk,),
        in_specs=[
            pl.BlockSpec((np_, bk), lambda k: (0, k)),
            pl.BlockSpec((bk, EMB_DIM), lambda k: (k, 0)),
            full(a_pcb), full(a_pcm), full(pri), full(cb), full(cm),
            full(m_pv), full(m_pcb), full(m_pcm), full(Wp), full(bp),
        ],
        out_specs=pl.BlockSpec((np_, EMB_DIM), lambda k: (0, 0)),
        out_shape=jax.ShapeDtypeStruct((np_, EMB_DIM), jnp.float32),
        scratch_shapes=[
            pltpu.VMEM((np_, EMB_DIM), jnp.float32),
            pltpu.VMEM((np_, 1), jnp.float32),
            pltpu.VMEM((np_, 1), jnp.float32),
        ],
        compiler_params=_VMEM_PARAMS,
    )(a_pv, item, a_pcb, a_pcm, pri, cb, cm, m_pv, m_pcb, m_pcm, Wp, bp)


def kernel(adjacency, adjacency_pv, adjacency_vp, adjacency_pcb,
           adjacency_cbp, adjacency_cbv, adjacency_vcb, adjacency_pcm,
           adjacency_cmp, adjacency_cmv, adjacency_vcm, adjacency_cbcm,
           adjacency_cmcb, item_emb, pri_emb, cateBig_emb, cateMiddle_emb,
           mat_vp, mat_vcb, mat_vcm, mat_pv, mat_pcb, mat_pcm, mat_cbp,
           mat_cbv, mat_cbcm, mat_cmp, mat_cmv, mat_cmcb, W_gi, b_gi,
           W_gp, b_gp, W_gcb, b_gcb, W_gcm, b_gcm):
    b_gi2 = b_gi.reshape(1, 1)
    b_gp2 = b_gp.reshape(1, 1)
    b_gcb2 = b_gcb.reshape(1, 1)
    b_gcm2 = b_gcm.reshape(1, 1)

    n = item_emb.shape[0]
    n_pad = (-n) % 2048
    pad_cols = lambda a: jnp.pad(a, ((0, 0), (0, n_pad)))
    a_pv_p = pad_cols(adjacency_pv)
    a_cbv_p = pad_cols(adjacency_cbv)
    a_cmv_p = pad_cols(adjacency_cmv)
    pad_rows = lambda a: jnp.pad(a, ((0, n_pad), (0, 0)))

    it1, q_adj, s_adj = _item_update_l1(
        adjacency, adjacency_vp, adjacency_vcb, adjacency_vcm,
        item_emb, pri_emb, cateBig_emb, cateMiddle_emb,
        mat_vp, mat_vcb, mat_vcm, W_gi, b_gi2)
    pr1, cb1, cm1 = _pcc_update(
        a_pv_p, adjacency_pcb, adjacency_pcm, adjacency_cbp,
        a_cbv_p, adjacency_cbcm, adjacency_cmp, a_cmv_p,
        adjacency_cmcb, pad_rows(item_emb), pri_emb, cateBig_emb,
        cateMiddle_emb, mat_pv, mat_pcb, mat_pcm, mat_cbp, mat_cbv,
        mat_cbcm, mat_cmp, mat_cmv, mat_cmcb, W_gp, b_gp2, W_gcb, b_gcb2,
        W_gcm, b_gcm2, n_pad)

    _ = (q_adj, s_adj)
    it2 = _item_update_f32(adjacency,
                           adjacency_vp, adjacency_vcb, adjacency_vcm,
                           it1, pr1, cb1, cm1, mat_vp, mat_vcb, mat_vcm,
                           W_gi, b_gi2)
    pr2 = _pr_update(a_pv_p, adjacency_pcb, adjacency_pcm,
                     pad_rows(it1), pr1, cb1, cm1, mat_pv, mat_pcb, mat_pcm,
                     W_gp, b_gp2, n_pad)
    return (it2, pr2)


# speedup vs baseline: 1.2829x; 1.2829x over previous
import functools

import jax
import jax.numpy as jnp
from jax.experimental import pallas as pl
from jax.experimental.pallas import tpu as pltpu

EMB_DIM = 128
_VMEM_PARAMS = pltpu.CompilerParams(vmem_limit_bytes=110 * 1024 * 1024)


def _intra_block(adj, mat, emb):
    r = jnp.sum(emb, axis=1)
    logits = mat * r[None, :]
    m = jnp.max(logits, axis=1, keepdims=True)
    e = jnp.exp(logits - m)
    s = e / jnp.sum(e, axis=1, keepdims=True)
    w = s * adj
    w = w / (jnp.sum(w, axis=1, keepdims=True) + 1e-8)
    return jnp.dot(w, emb, preferred_element_type=jnp.float32)


def _gate(e, W, b):
    return jnp.exp(jnp.sum(e * W, axis=1, keepdims=True) + b)


def _inter(W, b, e0, e1, e2, e3):
    g0 = _gate(e0, W, b)
    g1 = _gate(e1, W, b)
    g2 = _gate(e2, W, b)
    g3 = _gate(e3, W, b)
    s = g0 + g1 + g2 + g3
    return (g0 / s) * e0 + (g1 / s) * e1 + (g2 / s) * e2 + (g3 / s) * e3


def _item_kernel(adj_ref, avp_ref, avcb_ref, avcm_ref, item_ref, pri_ref,
                 cb_ref, cm_ref, mvp_ref, mvcb_ref, mvcm_ref, W_ref, b_ref,
                 out_ref, *, br):
    i = pl.program_id(0)
    e0 = item_ref[pl.ds(i * br, br), :]
    e1 = _intra_block(avp_ref[...], mvp_ref[...], pri_ref[...])
    e2 = _intra_block(avcb_ref[...], mvcb_ref[...], cb_ref[...])
    e3 = _intra_block(avcm_ref[...], mvcm_ref[...], cm_ref[...])
    gated = _inter(W_ref[...], b_ref[...], e0, e1, e2, e3)
    big = jnp.dot(adj_ref[...].astype(jnp.bfloat16),
                  item_ref[...].astype(jnp.bfloat16),
                  preferred_element_type=jnp.float32)
    out_ref[...] = gated + big


def _item_update(adjacency, a_vp, a_vcb, a_vcm, item, pri, cb, cm,
                 m_vp, m_vcb, m_vcm, W, b):
    n = item.shape[0]
    br = 400 if n % 400 == 0 else n
    grid = (n // br,)
    return pl.pallas_call(
        functools.partial(_item_kernel, br=br),
        grid=grid,
        in_specs=[
            pl.BlockSpec((br, n), lambda i: (i, 0)),
            pl.BlockSpec((br, a_vp.shape[1]), lambda i: (i, 0)),
            pl.BlockSpec((br, a_vcb.shape[1]), lambda i: (i, 0)),
            pl.BlockSpec((br, a_vcm.shape[1]), lambda i: (i, 0)),
            pl.BlockSpec((n, EMB_DIM), lambda i: (0, 0)),
            pl.BlockSpec(pri.shape, lambda i: (0, 0)),
            pl.BlockSpec(cb.shape, lambda i: (0, 0)),
            pl.BlockSpec(cm.shape, lambda i: (0, 0)),
            pl.BlockSpec((br, 1), lambda i: (i, 0)),
            pl.BlockSpec((br, 1), lambda i: (i, 0)),
            pl.BlockSpec((br, 1), lambda i: (i, 0)),
            pl.BlockSpec((1, EMB_DIM), lambda i: (0, 0)),
            pl.BlockSpec((1, 1), lambda i: (0, 0)),
        ],
        out_specs=pl.BlockSpec((br, EMB_DIM), lambda i: (i, 0)),
        out_shape=jax.ShapeDtypeStruct((n, EMB_DIM), jnp.float32),
        compiler_params=_VMEM_PARAMS,
    )(adjacency, a_vp, a_vcb, a_vcm, item, pri, cb, cm, m_vp, m_vcb, m_vcm,
      W, b)


def _chunks(n, w=2048):
    out = []
    off = 0
    while off < n:
        out.append((off, min(w, n - off)))
        off += w
    return out


def _stream_intra(adj_ref, mat, item_ref, n):
    num = None
    for off, w in _chunks(n):
        itb = item_ref[pl.ds(off, w), :]
        r = jnp.sum(itb, axis=1)
        e = jnp.exp(mat * r[None, :])
        wgt = e * adj_ref[:, pl.ds(off, w)]
        pnum = jnp.dot(wgt, itb, preferred_element_type=jnp.float32)
        ps = jnp.sum(wgt, axis=1, keepdims=True)
        pd = jnp.sum(e, axis=1, keepdims=True)
        if num is None:
            num, s, d = pnum, ps, pd
        else:
            num, s, d = num + pnum, s + ps, d + pd
    return num / (s + 1e-8 * d)


def _pcc_kernel(apv_ref, acbv_ref, acmv_ref, item_ref,
                apcb_ref, apcm_ref, acbp_ref, acbcm_ref, acmp_ref, acmcb_ref,
                pri_ref, cb_ref, cm_ref,
                mpv_ref, mcbv_ref, mcmv_ref, mpcb_ref, mpcm_ref, mcbp_ref,
                mcbcm_ref, mcmp_ref, mcmcb_ref,
                Wp_ref, bp_ref, Wcb_ref, bcb_ref, Wcm_ref, bcm_ref,
                pr_out, cbn_out, cmn_out, *, n):
    p = pri_ref[...]
    c_b = cb_ref[...]
    c_m = cm_ref[...]
    pr_out[...] = _inter(
        Wp_ref[...], bp_ref[...], p,
        _stream_intra(apv_ref, mpv_ref[...], item_ref, n),
        _intra_block(apcb_ref[...], mpcb_ref[...], c_b),
        _intra_block(apcm_ref[...], mpcm_ref[...], c_m))
    cbn_out[...] = _inter(
        Wcb_ref[...], bcb_ref[...], c_b,
        _intra_block(acbp_ref[...], mcbp_ref[...], p),
        _stream_intra(acbv_ref, mcbv_ref[...], item_ref, n),
        _intra_block(acbcm_ref[...], mcbcm_ref[...], c_m))
    cmn_out[...] = _inter(
        Wcm_ref[...], bcm_ref[...], c_m,
        _intra_block(acmp_ref[...], mcmp_ref[...], p),
        _stream_intra(acmv_ref, mcmv_ref[...], item_ref, n),
        _intra_block(acmcb_ref[...], mcmcb_ref[...], c_b))


def _pcc_update(a_pv, a_pcb, a_pcm, a_cbp, a_cbv, a_cbcm, a_cmp, a_cmv,
                a_cmcb, item, pri, cb, cm, m_pv, m_pcb, m_pcm, m_cbp, m_cbv,
                m_cbcm, m_cmp, m_cmv, m_cmcb, Wp, bp, Wcb, bcb, Wcm, bcm):
    n = item.shape[0]
    np_, ncb, ncm = pri.shape[0], cb.shape[0], cm.shape[0]
    return pl.pallas_call(
        functools.partial(_pcc_kernel, n=n),
        out_shape=(
            jax.ShapeDtypeStruct((np_, EMB_DIM), jnp.float32),
            jax.ShapeDtypeStruct((ncb, EMB_DIM), jnp.float32),
            jax.ShapeDtypeStruct((ncm, EMB_DIM), jnp.float32),
        ),
        compiler_params=_VMEM_PARAMS,
    )(a_pv, a_cbv, a_cmv, item, a_pcb, a_pcm, a_cbp, a_cbcm, a_cmp, a_cmcb,
      pri, cb, cm, m_pv, m_cbv, m_cmv, m_pcb, m_pcm, m_cbp, m_cbcm,
      m_cmp, m_cmcb, Wp, bp, Wcb, bcb, Wcm, bcm)


def _pr_kernel(apv_ref, item_ref, apcb_ref, apcm_ref, pri_ref, cb_ref,
               cm_ref, mpv_ref, mpcb_ref, mpcm_ref, Wp_ref, bp_ref, pr_out,
               *, n):
    pr_out[...] = _inter(
        Wp_ref[...], bp_ref[...], pri_ref[...],
        _stream_intra(apv_ref, mpv_ref[...], item_ref, n),
        _intra_block(apcb_ref[...], mpcb_ref[...], cb_ref[...]),
        _intra_block(apcm_ref[...], mpcm_ref[...], cm_ref[...]))


def _pr_update(a_pv, a_pcb, a_pcm, item, pri, cb, cm, m_pv, m_pcb, m_pcm,
               Wp, bp):
    n = item.shape[0]
    np_ = pri.shape[0]
    return pl.pallas_call(
        functools.partial(_pr_kernel, n=n),
        out_shape=jax.ShapeDtypeStruct((np_, EMB_DIM), jnp.float32),
        compiler_params=_VMEM_PARAMS,
    )(a_pv, item, a_pcb, a_pcm, pri, cb, cm, m_pv, m_pcb, m_pcm, Wp, bp)


def kernel(adjacency, adjacency_pv, adjacency_vp, adjacency_pcb,
           adjacency_cbp, adjacency_cbv, adjacency_vcb, adjacency_pcm,
           adjacency_cmp, adjacency_cmv, adjacency_vcm, adjacency_cbcm,
           adjacency_cmcb, item_emb, pri_emb, cateBig_emb, cateMiddle_emb,
           mat_vp, mat_vcb, mat_vcm, mat_pv, mat_pcb, mat_pcm, mat_cbp,
           mat_cbv, mat_cbcm, mat_cmp, mat_cmv, mat_cmcb, W_gi, b_gi,
           W_gp, b_gp, W_gcb, b_gcb, W_gcm, b_gcm):
    b_gi2 = b_gi.reshape(1, 1)
    b_gp2 = b_gp.reshape(1, 1)
    b_gcb2 = b_gcb.reshape(1, 1)
    b_gcm2 = b_gcm.reshape(1, 1)

    it1 = _item_update(adjacency, adjacency_vp, adjacency_vcb, adjacency_vcm,
                       item_emb, pri_emb, cateBig_emb, cateMiddle_emb,
                       mat_vp, mat_vcb, mat_vcm, W_gi, b_gi2)
    pr1, cb1, cm1 = _pcc_update(
        adjacency_pv, adjacency_pcb, adjacency_pcm, adjacency_cbp,
        adjacency_cbv, adjacency_cbcm, adjacency_cmp, adjacency_cmv,
        adjacency_cmcb, item_emb, pri_emb, cateBig_emb, cateMiddle_emb,
        mat_pv, mat_pcb, mat_pcm, mat_cbp, mat_cbv, mat_cbcm, mat_cmp,
        mat_cmv, mat_cmcb, W_gp, b_gp2, W_gcb, b_gcb2, W_gcm, b_gcm2)

    it2 = _item_update(adjacency, adjacency_vp, adjacency_vcb, adjacency_vcm,
                       it1, pr1, cb1, cm1, mat_vp, mat_vcb, mat_vcm,
                       W_gi, b_gi2)
    pr2 = _pr_update(adjacency_pv, adjacency_pcb, adjacency_pcm,
                     it1, pr1, cb1, cm1, mat_pv, mat_pcb, mat_pcm,
                     W_gp, b_gp2)
    return (it2, pr2)


# speedup vs baseline: 1.2978x; 1.0116x over previous
import functools

import jax
import jax.numpy as jnp
from jax.experimental import pallas as pl
from jax.experimental.pallas import tpu as pltpu

EMB_DIM = 128
_VMEM_PARAMS = pltpu.CompilerParams(vmem_limit_bytes=110 * 1024 * 1024)


def _intra_block(adj, mat, emb):
    r = jnp.sum(emb, axis=1)
    logits = mat * r[None, :]
    m = jnp.max(logits, axis=1, keepdims=True)
    e = jnp.exp(logits - m)
    s = e / jnp.sum(e, axis=1, keepdims=True)
    w = s * adj
    w = w / (jnp.sum(w, axis=1, keepdims=True) + 1e-8)
    return jnp.dot(w, emb, preferred_element_type=jnp.float32)


def _gate(e, W, b):
    return jnp.exp(jnp.sum(e * W, axis=1, keepdims=True) + b)


def _inter(W, b, e0, e1, e2, e3):
    g0 = _gate(e0, W, b)
    g1 = _gate(e1, W, b)
    g2 = _gate(e2, W, b)
    g3 = _gate(e3, W, b)
    s = g0 + g1 + g2 + g3
    return (g0 / s) * e0 + (g1 / s) * e1 + (g2 / s) * e2 + (g3 / s) * e3


def _item_kernel(adj0_ref, adj1_ref, avp_ref, avcb_ref, avcm_ref, item_ref,
                 pri_ref, cb_ref, cm_ref, mvp_ref, mvcb_ref, mvcm_ref,
                 W_ref, b_ref, out_ref, *, br):
    i = pl.program_id(0)
    e0 = item_ref[pl.ds(i * br, br), :]
    e1 = _intra_block(avp_ref[...], mvp_ref[...], pri_ref[...])
    e2 = _intra_block(avcb_ref[...], mvcb_ref[...], cb_ref[...])
    e3 = _intra_block(avcm_ref[...], mvcm_ref[...], cm_ref[...])
    gated = _inter(W_ref[...], b_ref[...], e0, e1, e2, e3)
    itb = item_ref[...].astype(jnp.bfloat16)
    h = br // 2
    big = jnp.concatenate([
        jnp.dot(adj0_ref[...].astype(jnp.bfloat16), itb,
                preferred_element_type=jnp.float32),
        jnp.dot(adj1_ref[...].astype(jnp.bfloat16), itb,
                preferred_element_type=jnp.float32),
    ], axis=0)
    out_ref[...] = gated + big


def _item_update(adjacency, a_vp, a_vcb, a_vcm, item, pri, cb, cm,
                 m_vp, m_vcb, m_vcm, W, b):
    n = item.shape[0]
    br = 400 if n % 400 == 0 else n
    hr = br // 2
    grid = (n // br,)
    return pl.pallas_call(
        functools.partial(_item_kernel, br=br),
        grid=grid,
        in_specs=[
            pl.BlockSpec((hr, n), lambda i: (2 * i, 0)),
            pl.BlockSpec((hr, n), lambda i: (2 * i + 1, 0)),
            pl.BlockSpec((br, a_vp.shape[1]), lambda i: (i, 0)),
            pl.BlockSpec((br, a_vcb.shape[1]), lambda i: (i, 0)),
            pl.BlockSpec((br, a_vcm.shape[1]), lambda i: (i, 0)),
            pl.BlockSpec((n, EMB_DIM), lambda i: (0, 0)),
            pl.BlockSpec(pri.shape, lambda i: (0, 0)),
            pl.BlockSpec(cb.shape, lambda i: (0, 0)),
            pl.BlockSpec(cm.shape, lambda i: (0, 0)),
            pl.BlockSpec((br, 1), lambda i: (i, 0)),
            pl.BlockSpec((br, 1), lambda i: (i, 0)),
            pl.BlockSpec((br, 1), lambda i: (i, 0)),
            pl.BlockSpec((1, EMB_DIM), lambda i: (0, 0)),
            pl.BlockSpec((1, 1), lambda i: (0, 0)),
        ],
        out_specs=pl.BlockSpec((br, EMB_DIM), lambda i: (i, 0)),
        out_shape=jax.ShapeDtypeStruct((n, EMB_DIM), jnp.float32),
        compiler_params=_VMEM_PARAMS,
    )(adjacency, adjacency, a_vp, a_vcb, a_vcm, item, pri, cb, cm,
      m_vp, m_vcb, m_vcm, W, b)


def _chunks(n, w=2048):
    out = []
    off = 0
    while off < n:
        out.append((off, min(w, n - off)))
        off += w
    return out


def _stream_intra(adj_ref, mat, item_ref, n):
    num = None
    for off, w in _chunks(n):
        itb = item_ref[pl.ds(off, w), :]
        r = jnp.sum(itb, axis=1)
        e = jnp.exp(mat * r[None, :])
        wgt = e * adj_ref[:, pl.ds(off, w)]
        pnum = jnp.dot(wgt, itb, preferred_element_type=jnp.float32)
        ps = jnp.sum(wgt, axis=1, keepdims=True)
        pd = jnp.sum(e, axis=1, keepdims=True)
        if num is None:
            num, s, d = pnum, ps, pd
        else:
            num, s, d = num + pnum, s + ps, d + pd
    return num / (s + 1e-8 * d)


def _pcc_kernel(apv_ref, acbv_ref, acmv_ref, item_ref,
                apcb_ref, apcm_ref, acbp_ref, acbcm_ref, acmp_ref, acmcb_ref,
                pri_ref, cb_ref, cm_ref,
                mpv_ref, mcbv_ref, mcmv_ref, mpcb_ref, mpcm_ref, mcbp_ref,
                mcbcm_ref, mcmp_ref, mcmcb_ref,
                Wp_ref, bp_ref, Wcb_ref, bcb_ref, Wcm_ref, bcm_ref,
                pr_out, cbn_out, cmn_out, *, n):
    p = pri_ref[...]
    c_b = cb_ref[...]
    c_m = cm_ref[...]
    pr_out[...] = _inter(
        Wp_ref[...], bp_ref[...], p,
        _stream_intra(apv_ref, mpv_ref[...], item_ref, n),
        _intra_block(apcb_ref[...], mpcb_ref[...], c_b),
        _intra_block(apcm_ref[...], mpcm_ref[...], c_m))
    cbn_out[...] = _inter(
        Wcb_ref[...], bcb_ref[...], c_b,
        _intra_block(acbp_ref[...], mcbp_ref[...], p),
        _stream_intra(acbv_ref, mcbv_ref[...], item_ref, n),
        _intra_block(acbcm_ref[...], mcbcm_ref[...], c_m))
    cmn_out[...] = _inter(
        Wcm_ref[...], bcm_ref[...], c_m,
        _intra_block(acmp_ref[...], mcmp_ref[...], p),
        _stream_intra(acmv_ref, mcmv_ref[...], item_ref, n),
        _intra_block(acmcb_ref[...], mcmcb_ref[...], c_b))


def _pcc_update(a_pv, a_pcb, a_pcm, a_cbp, a_cbv, a_cbcm, a_cmp, a_cmv,
                a_cmcb, item, pri, cb, cm, m_pv, m_pcb, m_pcm, m_cbp, m_cbv,
                m_cbcm, m_cmp, m_cmv, m_cmcb, Wp, bp, Wcb, bcb, Wcm, bcm):
    n = item.shape[0]
    np_, ncb, ncm = pri.shape[0], cb.shape[0], cm.shape[0]
    return pl.pallas_call(
        functools.partial(_pcc_kernel, n=n),
        out_shape=(
            jax.ShapeDtypeStruct((np_, EMB_DIM), jnp.float32),
            jax.ShapeDtypeStruct((ncb, EMB_DIM), jnp.float32),
            jax.ShapeDtypeStruct((ncm, EMB_DIM), jnp.float32),
        ),
        compiler_params=_VMEM_PARAMS,
    )(a_pv, a_cbv, a_cmv, item, a_pcb, a_pcm, a_cbp, a_cbcm, a_cmp, a_cmcb,
      pri, cb, cm, m_pv, m_cbv, m_cmv, m_pcb, m_pcm, m_cbp, m_cbcm,
      m_cmp, m_cmcb, Wp, bp, Wcb, bcb, Wcm, bcm)


def _pr_kernel(apv_ref, item_ref, apcb_ref, apcm_ref, pri_ref, cb_ref,
               cm_ref, mpv_ref, mpcb_ref, mpcm_ref, Wp_ref, bp_ref, pr_out,
               *, n):
    pr_out[...] = _inter(
        Wp_ref[...], bp_ref[...], pri_ref[...],
        _stream_intra(apv_ref, mpv_ref[...], item_ref, n),
        _intra_block(apcb_ref[...], mpcb_ref[...], cb_ref[...]),
        _intra_block(apcm_ref[...], mpcm_ref[...], cm_ref[...]))


def _pr_update(a_pv, a_pcb, a_pcm, item, pri, cb, cm, m_pv, m_pcb, m_pcm,
               Wp, bp):
    n = item.shape[0]
    np_ = pri.shape[0]
    return pl.pallas_call(
        functools.partial(_pr_kernel, n=n),
        out_shape=jax.ShapeDtypeStruct((np_, EMB_DIM), jnp.float32),
        compiler_params=_VMEM_PARAMS,
    )(a_pv, item, a_pcb, a_pcm, pri, cb, cm, m_pv, m_pcb, m_pcm, Wp, bp)


def kernel(adjacency, adjacency_pv, adjacency_vp, adjacency_pcb,
           adjacency_cbp, adjacency_cbv, adjacency_vcb, adjacency_pcm,
           adjacency_cmp, adjacency_cmv, adjacency_vcm, adjacency_cbcm,
           adjacency_cmcb, item_emb, pri_emb, cateBig_emb, cateMiddle_emb,
           mat_vp, mat_vcb, mat_vcm, mat_pv, mat_pcb, mat_pcm, mat_cbp,
           mat_cbv, mat_cbcm, mat_cmp, mat_cmv, mat_cmcb, W_gi, b_gi,
           W_gp, b_gp, W_gcb, b_gcb, W_gcm, b_gcm):
    b_gi2 = b_gi.reshape(1, 1)
    b_gp2 = b_gp.reshape(1, 1)
    b_gcb2 = b_gcb.reshape(1, 1)
    b_gcm2 = b_gcm.reshape(1, 1)

    it1 = _item_update(adjacency, adjacency_vp, adjacency_vcb, adjacency_vcm,
                       item_emb, pri_emb, cateBig_emb, cateMiddle_emb,
                       mat_vp, mat_vcb, mat_vcm, W_gi, b_gi2)
    pr1, cb1, cm1 = _pcc_update(
        adjacency_pv, adjacency_pcb, adjacency_pcm, adjacency_cbp,
        adjacency_cbv, adjacency_cbcm, adjacency_cmp, adjacency_cmv,
        adjacency_cmcb, item_emb, pri_emb, cateBig_emb, cateMiddle_emb,
        mat_pv, mat_pcb, mat_pcm, mat_cbp, mat_cbv, mat_cbcm, mat_cmp,
        mat_cmv, mat_cmcb, W_gp, b_gp2, W_gcb, b_gcb2, W_gcm, b_gcm2)

    it2 = _item_update(adjacency, adjacency_vp, adjacency_vcb, adjacency_vcm,
                       it1, pr1, cb1, cm1, mat_vp, mat_vcb, mat_vcm,
                       W_gi, b_gi2)
    pr2 = _pr_update(adjacency_pv, adjacency_pcb, adjacency_pcm,
                     it1, pr1, cb1, cm1, mat_pv, mat_pcb, mat_pcm,
                     W_gp, b_gp2)
    return (it2, pr2)
